# Initial kernel scaffold; baseline (speedup 1.0000x reference)
import jax
import jax.numpy as jnp
from jax.experimental import pallas as pl


def kernel(x0, x1, edge_index_00, edge_index_01, edge_index_10, edge_index_11, W1_00, W1_01, W1_10, W1_11, W2_00, W2_01, W2_10, W2_11, W3_00, W3_01, W3_10, W3_11, W4_00, W4_01, W4_10, W4_11):
    raise NotImplementedError("write your pallas kernel here")


# speedup vs baseline: 5.1587x; 5.1587x over previous
import functools

import jax
import jax.numpy as jnp
from jax import lax
from jax.experimental import pallas as pl
from jax.experimental.pallas import tpu as pltpu
from jax.experimental.pallas import tpu_sc as plsc

N = 10000
E = 320000
D = 128
H1 = 64
H2 = 32

NC = 2
NS = 16
NW = NC * NS
EPW = E // NW
CHUNK = 128
NCHUNK = 80
EPW_PAD = NCHUNK * CHUNK
ACC_ROWS = 10240
RPT = ACC_ROWS // NS
ZROWS = 128
DW = 16


def _prep_edges(ei):
    src = ei[0].reshape(NW, EPW)
    dst = ei[1].reshape(NW, EPW)
    pad = EPW_PAD - EPW
    src = jnp.pad(src, ((0, 0), (0, pad)), constant_values=0)
    dst = jnp.pad(dst, ((0, 0), (0, pad)), constant_values=N)
    return (src.reshape(NW, NCHUNK, CHUNK), dst.reshape(NW, NCHUNK, CHUNK))


_MESH = plsc.VectorSubcoreMesh(
    core_axis_name="c", subcore_axis_name="s", num_cores=NC, num_subcores=NS)
_SC_PARAMS = pltpu.CompilerParams(use_tc_tiling_on_sc=False)


def _make_deg_kernel():
    @functools.partial(
        pl.kernel,
        out_type=jax.ShapeDtypeStruct((4 * NC, ACC_ROWS, DW), jnp.float32),
        mesh=_MESH,
        scratch_types=[
            pltpu.VMEM_SHARED((ACC_ROWS, DW), jnp.float32),
            pltpu.VMEM_SHARED((ACC_ROWS, DW), jnp.float32),
            pltpu.VMEM_SHARED((ACC_ROWS, DW), jnp.float32),
            pltpu.VMEM_SHARED((ACC_ROWS, DW), jnp.float32),
            pltpu.VMEM((NCHUNK, CHUNK), jnp.int32),
            pltpu.VMEM((CHUNK, DW), jnp.float32),
            pltpu.VMEM((ZROWS, DW), jnp.float32),
        ],
        compiler_params=_SC_PARAMS,
    )
    def deg_kernel(ones_h, zeros_h, d0, d1, d2, d3, out,
                   acc0, acc1, acc2, acc3, dst_v, ones_v, zeros_v):
        cid = lax.axis_index("c")
        sid = lax.axis_index("s")
        wid = cid * NS + sid
        pltpu.sync_copy(ones_h, ones_v)
        pltpu.sync_copy(zeros_h, zeros_v)
        dsts = [d0, d1, d2, d3]
        accs = [acc0, acc1, acc2, acc3]
        for r in range(4):
            for z in range(RPT // ZROWS):
                pltpu.sync_copy(
                    zeros_v, accs[r].at[pl.ds(sid * RPT + z * ZROWS, ZROWS)])
        for r in range(4):
            pltpu.sync_copy(dsts[r].at[wid], dst_v)
            plsc.subcore_barrier()

            @pl.loop(0, NCHUNK)
            def _(j):
                pltpu.sync_copy(ones_v, accs[r].at[dst_v.at[j]], add=True)

            plsc.subcore_barrier()
            pltpu.sync_copy(accs[r].at[pl.ds(sid * RPT, RPT)],
                            out.at[r * NC + cid, pl.ds(sid * RPT, RPT)])

    return deg_kernel


def _make_conv_kernel(h):
    @functools.partial(
        pl.kernel,
        out_type=jax.ShapeDtypeStruct((4 * NC, ACC_ROWS, h), jnp.float32),
        mesh=_MESH,
        scratch_types=[
            pltpu.VMEM_SHARED((ACC_ROWS, h), jnp.float32),
            pltpu.VMEM((NCHUNK, CHUNK), jnp.int32),
            pltpu.VMEM((NCHUNK, CHUNK), jnp.int32),
            pltpu.VMEM((CHUNK, h), jnp.float32),
            pltpu.VMEM((ZROWS, h), jnp.float32),
            pltpu.SemaphoreType.DMA,
        ],
        compiler_params=_SC_PARAMS,
    )
    def conv_kernel(zeros_h, t0, t1, t2, t3, s0, s1, s2, s3, d0, d1, d2, d3,
                    out, acc_sp, src_v, dst_v, rows_v, zeros_v, sem):
        cid = lax.axis_index("c")
        sid = lax.axis_index("s")
        wid = cid * NS + sid
        pltpu.sync_copy(zeros_h, zeros_v)
        tables = [t0, t1, t2, t3]
        srcs = [s0, s1, s2, s3]
        dsts = [d0, d1, d2, d3]
        for r in range(4):
            for z in range(RPT // ZROWS):
                pltpu.sync_copy(
                    zeros_v, acc_sp.at[pl.ds(sid * RPT + z * ZROWS, ZROWS)])
            pltpu.sync_copy(srcs[r].at[wid], src_v)
            pltpu.sync_copy(dsts[r].at[wid], dst_v)
            plsc.subcore_barrier()

            @pl.loop(0, NCHUNK)
            def _(j):
                pltpu.async_copy(tables[r].at[src_v.at[j]], rows_v, sem).wait()
                pltpu.sync_copy(rows_v, acc_sp.at[dst_v.at[j]], add=True)

            plsc.subcore_barrier()
            pltpu.sync_copy(acc_sp.at[pl.ds(sid * RPT, RPT)],
                            out.at[r * NC + cid, pl.ds(sid * RPT, RPT)])

    return conv_kernel


_deg_call = _make_deg_kernel()
_conv64 = _make_conv_kernel(H1)
_conv32 = _make_conv_kernel(H2)


RB = 2000
_NG = N // RB


def _row_spec(width):
    return pl.BlockSpec((RB, width), lambda i: (i, 0))


def _full_spec(a, b):
    return pl.BlockSpec((a, b), lambda i: (0, 0))


def _agg_spec(width):
    return pl.BlockSpec((4 * NC, RB, width), lambda i: (0, i, 0))


def _proj1(x0, x1, w00, w01, w10, w11):
    def body(x0_r, x1_r, w00_r, w01_r, w10_r, w11_r, p00, p01, p10, p11):
        p00[...] = jnp.dot(x0_r[...], w00_r[...],
                           preferred_element_type=jnp.float32)
        p01[...] = jnp.dot(x1_r[...], w01_r[...],
                           preferred_element_type=jnp.float32)
        p10[...] = jnp.dot(x0_r[...], w10_r[...],
                           preferred_element_type=jnp.float32)
        p11[...] = jnp.dot(x1_r[...], w11_r[...],
                           preferred_element_type=jnp.float32)

    return pl.pallas_call(
        body,
        grid=(_NG,),
        in_specs=[_row_spec(D), _row_spec(D)] + [_full_spec(D, H1)] * 4,
        out_specs=[_row_spec(H1)] * 4,
        out_shape=[jax.ShapeDtypeStruct((N, H1), jnp.float32)] * 4,
    )(x0, x1, w00, w01, w10, w11)


def _norm_pair(agg_r, deg_r, r):
    a = agg_r[r * NC] + agg_r[r * NC + 1]
    d = deg_r[r * NC, :, 0:1] + deg_r[r * NC + 1, :, 0:1]
    return a / jnp.maximum(d, 1.0)


def _combine_proj(h_in, h_out, agg, deg, w00, w01, w10, w11):
    def body(agg_r, deg_r, w00_r, w01_r, w10_r, w11_r,
             h0_r, h1_r, p00, p01, p10, p11):
        h0 = jax.nn.relu(_norm_pair(agg_r, deg_r, 0)
                         + _norm_pair(agg_r, deg_r, 1))
        h1 = jax.nn.relu(_norm_pair(agg_r, deg_r, 2)
                         + _norm_pair(agg_r, deg_r, 3))
        h0_r[...] = h0
        h1_r[...] = h1
        p00[...] = jnp.dot(h0, w00_r[...], preferred_element_type=jnp.float32)
        p01[...] = jnp.dot(h1, w01_r[...], preferred_element_type=jnp.float32)
        p10[...] = jnp.dot(h0, w10_r[...], preferred_element_type=jnp.float32)
        p11[...] = jnp.dot(h1, w11_r[...], preferred_element_type=jnp.float32)

    return pl.pallas_call(
        body,
        grid=(_NG,),
        in_specs=([_agg_spec(h_in), _agg_spec(DW)]
                  + [_full_spec(h_in, h_out)] * 4),
        out_specs=[_row_spec(h_in)] * 2 + [_row_spec(h_out)] * 4,
        out_shape=(
            [jax.ShapeDtypeStruct((N, h_in), jnp.float32)] * 2
            + [jax.ShapeDtypeStruct((N, h_out), jnp.float32)] * 4
        ),
    )(agg, deg, w00, w01, w10, w11)


def _final_combine(agg, deg):
    def body(agg_r, deg_r, e0_r, e1_r):
        e0_r[...] = _norm_pair(agg_r, deg_r, 0) + _norm_pair(agg_r, deg_r, 1)
        e1_r[...] = _norm_pair(agg_r, deg_r, 2) + _norm_pair(agg_r, deg_r, 3)

    return pl.pallas_call(
        body,
        grid=(_NG,),
        in_specs=[_agg_spec(H2), _agg_spec(DW)],
        out_specs=[_row_spec(H2)] * 2,
        out_shape=[jax.ShapeDtypeStruct((N, H2), jnp.float32)] * 2,
    )(agg, deg)


def kernel(x0, x1, edge_index_00, edge_index_01, edge_index_10, edge_index_11,
           W1_00, W1_01, W1_10, W1_11,
           W2_00, W2_01, W2_10, W2_11,
           W3_00, W3_01, W3_10, W3_11,
           W4_00, W4_01, W4_10, W4_11):
    s00, d00 = _prep_edges(edge_index_00)
    s01, d01 = _prep_edges(edge_index_01)
    s10, d10 = _prep_edges(edge_index_10)
    s11, d11 = _prep_edges(edge_index_11)
    srcs = (s00, s01, s10, s11)
    dsts = (d00, d01, d10, d11)
    ones_hd = jnp.ones((CHUNK, DW), jnp.float32)
    zeros_hd = jnp.zeros((ZROWS, DW), jnp.float32)
    zeros_h64 = jnp.zeros((ZROWS, H1), jnp.float32)
    zeros_h32 = jnp.zeros((ZROWS, H2), jnp.float32)

    deg = _deg_call(ones_hd, zeros_hd, *dsts)

    p1 = _proj1(x0, x1, W1_00, W1_01, W1_10, W1_11)
    agg1 = _conv64(zeros_h64, *p1, *srcs, *dsts)
    h1_0, h1_1, *p2 = _combine_proj(H1, H2, agg1, deg,
                                    W2_00, W2_01, W2_10, W2_11)
    agg2 = _conv32(zeros_h32, *p2, *srcs, *dsts)
    e1_0, e1_1, *p3 = _combine_proj(H2, H2, agg2, deg,
                                    W3_00, W3_01, W3_10, W3_11)
    agg3 = _conv32(zeros_h32, *p3, *srcs, *dsts)
    _, _, *p4 = _combine_proj(H2, H2, agg3, deg,
                              W4_00, W4_01, W4_10, W4_11)
    agg4 = _conv32(zeros_h32, *p4, *srcs, *dsts)
    e4_0, e4_1 = _final_combine(agg4, deg)

    out0 = jnp.concatenate([h1_0, e1_0, e4_0], axis=1)
    out1 = jnp.concatenate([h1_1, e1_1, e4_1], axis=1)
    return jnp.concatenate([out0, out1], axis=0)
